# Initial kernel scaffold; baseline (speedup 1.0000x reference)
import jax
import jax.numpy as jnp
from jax.experimental import pallas as pl


def kernel(entity_mod, entity_phase, entity_radius, relation_mod, relation_phase, relation_bias, relation_radius, mod_weight, phase_weight, head_idx, rel_idx, neg_idx):
    raise NotImplementedError("write your pallas kernel here")


# speedup vs baseline: 9.1418x; 9.1418x over previous
import functools

import jax
import jax.numpy as jnp
from jax import lax
from jax.experimental import pallas as pl
from jax.experimental.pallas import tpu as pltpu
from jax.experimental.pallas import tpu_sc as plsc

GAMMA = 24.0
EPSILON = 2.0
PI = 3.1415926235897933
CEN = 0.02

_SIN_C = (0.9999673025915434, -0.1666224038748874, 0.008316284383106321,
          -0.0001955960029411126, 2.527388131797967e-06,
          -1.588366235760752e-08)


def _sin_poly(x):
    x2 = x * x
    p = jnp.float32(_SIN_C[5])
    for c in _SIN_C[4::-1]:
        p = p * x2 + jnp.float32(c)
    return x * p

CH = 64
NLANE = 16


def _sc_gather(nw, tpw, hpw, dim, ne, nr,
               emod, ephase, erad, rmod, rphase, rbias, rrad,
               hidx, ridx, nidx):
    nneg_rows = nidx.shape[0]
    nb = hidx.shape[0]
    n_chunks = tpw // CH
    h_chunks = hpw // CH
    f32 = jnp.float32

    mesh = plsc.VectorSubcoreMesh(core_axis_name="c", subcore_axis_name="s")

    out_type = [
        jax.ShapeDtypeStruct((nneg_rows, dim), f32),
        jax.ShapeDtypeStruct((nneg_rows, dim), f32),
        jax.ShapeDtypeStruct((nneg_rows,), f32),
        jax.ShapeDtypeStruct((nb, dim), f32),
        jax.ShapeDtypeStruct((nb, dim), f32),
        jax.ShapeDtypeStruct((nb,), f32),
        jax.ShapeDtypeStruct((nb, dim), f32),
        jax.ShapeDtypeStruct((nb, dim), f32),
        jax.ShapeDtypeStruct((nb, dim), f32),
        jax.ShapeDtypeStruct((nb,), f32),
    ]

    @functools.partial(
        pl.kernel,
        out_type=out_type,
        mesh=mesh,
        compiler_params=pltpu.CompilerParams(needs_layout_passes=False),
        scratch_types=[
            pltpu.VMEM((ne,), f32),
            pltpu.VMEM((nr,), f32),
            pltpu.VMEM((CH,), jnp.int32),
            pltpu.VMEM((CH, dim), f32),
            pltpu.VMEM((CH, dim), f32),
            pltpu.VMEM((CH,), f32),
            pltpu.SemaphoreType.DMA,
            pltpu.SemaphoreType.DMA,
        ],
    )
    def gather_kernel(emod_h, ephase_h, erad_h, rmod_h, rphase_h, rbias_h,
                      rrad_h, hidx_h, ridx_h, nidx_h,
                      tmod_o, tphase_o, trad_o, hmod_o, hphase_o, hrad_o,
                      rmod_o, rphase_o, rbias_o, rrad_o,
                      erad_v, rrad_v, idx_v, rows_a, rows_b, radc_v, sa, sb):
        wid = lax.axis_index("s") * 2 + lax.axis_index("c")
        pltpu.sync_copy(erad_h, erad_v)
        pltpu.sync_copy(rrad_h, rrad_v)

        def rad_gather(tab_v):
            for k in range(CH // NLANE):
                iv = idx_v[pl.ds(k * NLANE, NLANE)]
                radc_v[pl.ds(k * NLANE, NLANE)] = plsc.load_gather(tab_v, [iv])

        def row_chunk(off, idx_h, tab1_h, tab2_h, radtab_v,
                      out1_o, out2_o, radout_o):
            pltpu.sync_copy(idx_h.at[pl.ds(off, CH)], idx_v)
            ca = pltpu.async_copy(tab1_h.at[idx_v], rows_a, sa)
            cb = pltpu.async_copy(tab2_h.at[idx_v], rows_b, sb)
            rad_gather(radtab_v)
            ca.wait()
            cb.wait()
            pltpu.sync_copy(rows_a, out1_o.at[pl.ds(off, CH)])
            pltpu.sync_copy(rows_b, out2_o.at[pl.ds(off, CH)])
            pltpu.sync_copy(radc_v, radout_o.at[pl.ds(off, CH)])

        tbase = wid * tpw

        def tail_chunk(c, carry):
            row_chunk(tbase + c * CH, nidx_h, emod_h, ephase_h, erad_v,
                      tmod_o, tphase_o, trad_o)
            return carry

        lax.fori_loop(0, n_chunks, tail_chunk, 0)

        hbase = wid * hpw
        for c in range(h_chunks):
            row_chunk(hbase + c * CH, hidx_h, emod_h, ephase_h, erad_v,
                      hmod_o, hphase_o, hrad_o)

        for c in range(h_chunks):
            off = hbase + c * CH
            row_chunk(off, ridx_h, rmod_h, rphase_h, rrad_v,
                      rmod_o, rphase_o, rrad_o)
            cb = pltpu.async_copy(rbias_h.at[idx_v], rows_b, sb)
            cb.wait()
            pltpu.sync_copy(rows_b, rbias_o.at[pl.ds(off, CH)])

    return gather_kernel(emod, ephase, erad, rmod, rphase, rbias, rrad,
                         hidx, ridx, nidx)


def _tc_score(erange, hmod, hphase, hrad, rmodg, rphaseg, rbiasg, rradg,
              mod_weight, phase_weight, tmod, tphase, trad):
    b, nneg, dim = tmod.shape
    bq = 32
    inv = PI / erange
    inv_er = 1.0 / erange

    def body(mw_r, pw_r, hmod_r, hphase_r, hrad_r, rmod_r, rphase_r,
             rbias_r, rrad_r, tmod_r, tphase_r, trad_r, out_r):
        mw = mw_r[0, 0]
        pw = pw_r[0, 0]
        rm = jnp.abs(rmod_r[...])
        rb = jnp.minimum(rbias_r[...], 1.0)
        rb = jnp.where(rb < -rm, -rm, rb)
        mod_e = hmod_r[...] * (rm + rb)
        phase_e = (hphase_r[...] + rphase_r[...]) * inv
        rad_e = jnp.abs(hrad_r[...] * inv_er) * jnp.abs(rrad_r[...])
        scale = 1.0 - rb

        md = mod_e[:, None, :] - tmod_r[...] * scale[:, None, :]
        mod_dist = jnp.sqrt(jnp.sum(md * md, axis=-1))
        pd = phase_e[:, None, :] * 0.5 - tphase_r[...] * (inv * 0.5)
        phase_dist = jnp.sum(jnp.abs(_sin_poly(pd)), axis=-1)
        rad_dist = jnp.abs(rad_e + jnp.abs(trad_r[...] * inv_er))
        out_r[...] = GAMMA - (mw * mod_dist + pw * phase_dist
                              - CEN * rad_dist)

    smem = pl.BlockSpec(memory_space=pltpu.SMEM)
    return pl.pallas_call(
        body,
        grid=(b // bq,),
        in_specs=[
            smem,
            smem,
            pl.BlockSpec((bq, dim), lambda i: (i, 0)),
            pl.BlockSpec((bq, dim), lambda i: (i, 0)),
            pl.BlockSpec((bq, 1), lambda i: (i, 0)),
            pl.BlockSpec((bq, dim), lambda i: (i, 0)),
            pl.BlockSpec((bq, dim), lambda i: (i, 0)),
            pl.BlockSpec((bq, dim), lambda i: (i, 0)),
            pl.BlockSpec((bq, 1), lambda i: (i, 0)),
            pl.BlockSpec((bq, nneg, dim), lambda i: (i, 0, 0)),
            pl.BlockSpec((bq, nneg, dim), lambda i: (i, 0, 0)),
            pl.BlockSpec((bq, nneg), lambda i: (i, 0)),
        ],
        out_specs=pl.BlockSpec((bq, nneg), lambda i: (i, 0)),
        out_shape=jax.ShapeDtypeStruct((b, nneg), jnp.float32),
    )(mod_weight, phase_weight, hmod, hphase, hrad, rmodg, rphaseg,
      rbiasg, rradg, tmod, tphase, trad)


def kernel(entity_mod, entity_phase, entity_radius, relation_mod,
           relation_phase, relation_bias, relation_radius, mod_weight,
           phase_weight, head_idx, rel_idx, neg_idx):
    b, nneg = neg_idx.shape
    dim = entity_mod.shape[1]
    ne = entity_mod.shape[0]
    nr = relation_mod.shape[0]
    erange = (GAMMA + EPSILON) / dim

    nw = 32
    tpw = (b * nneg) // nw
    hpw = b // nw
    assert tpw % CH == 0 and hpw % CH == 0

    erad = entity_radius.reshape(-1)
    rrad = relation_radius.reshape(-1)
    nidx = neg_idx.reshape(-1)

    (tmod, tphase, trad, hmod, hphase, hrad,
     rmodg, rphaseg, rbiasg, rradg) = _sc_gather(
        nw, tpw, hpw, dim, ne, nr,
        entity_mod, entity_phase, erad, relation_mod, relation_phase,
        relation_bias, rrad, head_idx, rel_idx, nidx)

    tmod = tmod.reshape(b, nneg, dim)
    tphase = tphase.reshape(b, nneg, dim)
    trad = trad.reshape(b, nneg)
    hrad = hrad[:, None]
    rradg = rradg[:, None]

    return _tc_score(erange, hmod, hphase, hrad, rmodg, rphaseg, rbiasg,
                     rradg, mod_weight, phase_weight, tmod, tphase, trad)
